# Initial kernel scaffold; baseline (speedup 1.0000x reference)
import jax
import jax.numpy as jnp
from jax.experimental import pallas as pl


def kernel(ref_pts, ctr_coor, ctr_reg):
    raise NotImplementedError("write your pallas kernel here")


# speedup vs baseline: 193.8555x; 193.8555x over previous
import functools
import math

import jax
import jax.numpy as jnp
import numpy as np
from jax import lax
from jax.experimental import pallas as pl
from jax.experimental.pallas import tpu as pltpu
from jax.experimental.pallas import tpu_sc as plsc

B = 4
SIZE = 128
PAD = 2
S = SIZE + 2 * PAD
CELLS = B * S * S
NW = 32
CPT = 2304
OUT_CELLS = NW * CPT
GCH = 128
CH = 2048


def _pad1d(a, n, val):
    return jnp.concatenate([a, jnp.full((n - a.shape[0],), val, a.dtype)])


def _stage_a(nvox, nrows):
    mesh = plsc.VectorSubcoreMesh(core_axis_name="c", subcore_axis_name="s")

    @functools.partial(
        pl.kernel,
        mesh=mesh,
        compiler_params=pltpu.CompilerParams(
            needs_layout_passes=False, use_tc_tiling_on_sc=False),
        out_type=jax.ShapeDtypeStruct((OUT_CELLS, 8), jnp.float32),
        scratch_types=[
            pltpu.VMEM((CPT,), jnp.int32),
            pltpu.VMEM((CPT, 8), jnp.float32),
            pltpu.VMEM((CH,), jnp.int32),
            pltpu.VMEM((CH,), jnp.int32),
            pltpu.VMEM((CH,), jnp.int32),
            pltpu.SemaphoreType.DMA,
        ],
    )
    def stage_a(cb_hbm, cx_hbm, cy_hbm, tab_hbm, out_hbm,
                map_v, rows_v, cb_v, cx_v, cy_v, sem):
        wid = lax.axis_index("s") * 2 + lax.axis_index("c")
        lo = wid * CPT
        lane = lax.iota(jnp.int32, 16)

        def init_body(i, _):
            map_v[pl.ds(pl.multiple_of(i * 16, 16), 16)] = jnp.full(
                (16,), -1, jnp.int32)
            return 0
        lax.fori_loop(0, CPT // 16, init_body, 0)

        def chunk_body(k, _):
            off = pl.multiple_of(k * CH, CH)
            pltpu.sync_copy(cb_hbm.at[pl.ds(off, CH)], cb_v)
            pltpu.sync_copy(cx_hbm.at[pl.ds(off, CH)], cx_v)
            pltpu.sync_copy(cy_hbm.at[pl.ds(off, CH)], cy_v)

            def grp_body(g, _):
                goff = pl.ds(pl.multiple_of(g * 16, 16), 16)
                b = cb_v[goff]
                x = cx_v[goff]
                y = cy_v[goff]
                c = b * (S * S) + ((x >> 1) + PAD) * S + ((y >> 1) + PAD)
                val = k * CH + g * 16 + lane
                mine = (c >= lo) & (c < lo + CPT)
                idx = jnp.where(mine, c - lo, 0)
                plsc.store_scatter(map_v, [idx], val, mask=mine)
                for _ in range(3):
                    w = plsc.load_gather(map_v, [idx])
                    redo = mine & (val > w)
                    plsc.store_scatter(map_v, [idx], val, mask=redo)
                return 0
            lax.fori_loop(0, CH // 16, grp_body, 0)
            return 0
        lax.fori_loop(0, nvox // CH, chunk_body, 0)

        def fill_body(i, _):
            ioff = pl.ds(pl.multiple_of(i * 16, 16), 16)
            m = map_v[ioff]
            map_v[ioff] = jnp.where(m < 0, (nrows - 8) + (lane & 7), m)
            return 0
        lax.fori_loop(0, CPT // 16, fill_body, 0)

        copies = []
        for j in range(CPT // GCH):
            copies.append(pltpu.async_copy(
                tab_hbm.at[map_v.at[pl.ds(j * GCH, GCH)]],
                rows_v.at[pl.ds(j * GCH, GCH)], sem))
        for cp in copies:
            cp.wait()
        pltpu.sync_copy(rows_v, out_hbm.at[pl.ds(lo, CPT)])

    return stage_a


def _stencil_body(vcx, vcy, r0, r1, r2, r3, r4, r5, e0, e1):
    def pos(axis):
        n = 4 * lax.broadcasted_iota(jnp.int32, (B, SIZE, SIZE), axis) + 4
        return ((n * 13108) >> 16).astype(jnp.float32)
    px = pos(1)
    py = pos(2)
    acc0 = jnp.zeros((B, SIZE, SIZE), jnp.float32)
    acc1 = jnp.zeros((B, SIZE, SIZE), jnp.float32)
    for dx in range(3):
        for dy in range(3):
            sl = (slice(None), slice(1 + dx, 1 + dx + SIZE),
                  slice(1 + dy, 1 + dy + SIZE))
            vx = (vcx[sl] + 0.5) * 0.4
            vy = (vcy[sl] + 0.5) * 0.4
            d0 = px - vx
            d1 = py - vy
            v00 = jnp.maximum(r2[sl], 0.0) + 0.1
            v01 = jnp.maximum(r3[sl], 0.0) + 0.1
            v10 = jnp.maximum(r4[sl], 0.0) + 0.1
            v11 = jnp.maximum(r5[sl], 0.0) + 0.1
            d0s = d0 * d0
            d1s = d1 * d1
            acc0 += jnp.exp(-0.5 * (d0s / v00 + d1s / v01)) * \
                jnp.maximum(r0[sl], 0.0)
            acc1 += jnp.exp(-0.5 * (d0s / v10 + d1s / v11)) * \
                jnp.maximum(r1[sl], 0.0)
    e0[...] = acc0
    e1[...] = acc1


def _stage_c(qpad):
    qpt = qpad // NW
    mesh = plsc.VectorSubcoreMesh(core_axis_name="c", subcore_axis_name="s")

    @functools.partial(
        pl.kernel,
        mesh=mesh,
        compiler_params=pltpu.CompilerParams(needs_layout_passes=False),
        out_type=[jax.ShapeDtypeStruct((qpad,), jnp.float32),
                  jax.ShapeDtypeStruct((qpad,), jnp.float32)],
        scratch_types=[
            pltpu.VMEM((B * SIZE * SIZE,), jnp.float32),
            pltpu.VMEM((qpt,), jnp.int32),
            pltpu.VMEM((qpt,), jnp.int32),
            pltpu.VMEM((qpt,), jnp.int32),
            pltpu.VMEM((qpt,), jnp.float32),
        ],
    )
    def stage_c(qb_hbm, qx_hbm, qy_hbm, e0_hbm, e1_hbm, o0_hbm, o1_hbm,
                ev, qb_v, qx_v, qy_v, ov):
        wid = lax.axis_index("s") * 2 + lax.axis_index("c")
        qlo = wid * qpt
        pltpu.sync_copy(qb_hbm.at[pl.ds(qlo, qpt)], qb_v)
        pltpu.sync_copy(qx_hbm.at[pl.ds(qlo, qpt)], qx_v)
        pltpu.sync_copy(qy_hbm.at[pl.ds(qlo, qpt)], qy_v)

        def one_channel(e_hbm, o_hbm):
            pltpu.sync_copy(e_hbm, ev)

            def body(g, _):
                goff = pl.ds(pl.multiple_of(g * 16, 16), 16)
                b = qb_v[goff]
                x = qx_v[goff]
                y = qy_v[goff]
                cx = (5 * x) >> 2
                cy = (5 * y) >> 2
                idx = b * (SIZE * SIZE) + cx * SIZE + cy
                ov[goff] = plsc.load_gather(ev, [idx])
                return 0
            lax.fori_loop(0, qpt // 16, body, 0)
            pltpu.sync_copy(ov, o_hbm.at[pl.ds(qlo, qpt)])

        one_channel(e0_hbm, o0_hbm)
        one_channel(e1_hbm, o1_hbm)

    return stage_c


def kernel(ref_pts, ctr_coor, ctr_reg):
    nq = ref_pts.shape[0]
    nv = ctr_coor.shape[0]
    nvox = ((nv + CH - 1) // CH) * CH
    qpt = ((nq + NW - 1) // NW + 15) // 16 * 16
    qpad = qpt * NW
    nrows = nv + 8

    cb = _pad1d(ctr_coor[:, 0], nvox, B)
    cx = _pad1d(ctr_coor[:, 1], nvox, 0)
    cy = _pad1d(ctr_coor[:, 2], nvox, 0)
    sent = jnp.tile(jnp.array([[0., 0., -1., -1., -1., -1., -1., -1.]],
                              jnp.float32), (8, 1))
    tab = jnp.concatenate(
        [jnp.concatenate([ctr_coor[:, 1:3].astype(jnp.float32), ctr_reg],
                         axis=1), sent], axis=0)

    gath = _stage_a(nvox, nrows)(cb, cx, cy, tab)
    planes = gath[:CELLS].T.reshape(8, B, S, S)

    e0, e1 = pl.pallas_call(
        _stencil_body,
        out_shape=[jax.ShapeDtypeStruct((B, SIZE, SIZE), jnp.float32)] * 2,
    )(*[planes[i] for i in range(8)])

    qb = _pad1d(ref_pts[:, 0], qpad, 0)
    qx = _pad1d(ref_pts[:, 1], qpad, 0)
    qy = _pad1d(ref_pts[:, 2], qpad, 0)
    o0, o1 = _stage_c(qpad)(qb, qx, qy, e0.reshape(-1), e1.reshape(-1))
    return jnp.stack([o0[:nq], o1[:nq]], axis=1)
